# Initial kernel scaffold; baseline (speedup 1.0000x reference)
import jax
import jax.numpy as jnp
from jax.experimental import pallas as pl


def kernel(x, edge_index, W1, b1, W2, b2, W3, b3):
    raise NotImplementedError("write your pallas kernel here")


# speedup vs baseline: 7.4881x; 7.4881x over previous
import functools

import jax
import jax.numpy as jnp
from jax import lax
from jax.experimental import pallas as pl
from jax.experimental.pallas import tpu as pltpu
from jax.experimental.pallas import tpu_sc as plsc

N = 10000
NPAD = 10240
E = 160000
TILES = 16
K = 128
CH = 80
EPAD = TILES * CH * K
RPT = NPAD // TILES
NBUF = 2
F = 128
R = 512
GRID = NPAD // R

_f32 = jnp.float32


def _mesh():
  return plsc.VectorSubcoreMesh(
      core_axis_name="c", subcore_axis_name="s", num_cores=2,
      num_subcores=TILES)


def _agg_run(u_hbm, s_out, z, dsts, src_v, dstr, acc, rows, gsems, dsems, s):
  r0 = s * RPT
  pltpu.sync_copy(z.at[pl.ds(r0, RPT)], acc.at[pl.ds(r0, RPT)])
  plsc.subcore_barrier()
  for b in range(NBUF):
    pltpu.async_copy(u_hbm.at[src_v.at[b]], rows.at[b], gsems[b])
    pltpu.async_copy(dsts.at[s].at[b], dstr.at[b], dsems[b])

  def outer(t, carry):
    for b in range(NBUF):
      j = t * NBUF + b
      pltpu.make_async_copy(u_hbm.at[src_v.at[j]], rows.at[b],
                            gsems[b]).wait()
      pltpu.make_async_copy(dsts.at[s].at[j], dstr.at[b], dsems[b]).wait()
      pltpu.sync_copy(rows.at[b], acc.at[dstr.at[b]], add=True)
      jn = j + NBUF

      @pl.when(jn < CH)
      def _fire():
        pltpu.async_copy(u_hbm.at[src_v.at[jn]], rows.at[b], gsems[b])
        pltpu.async_copy(dsts.at[s].at[jn], dstr.at[b], dsems[b])
    return carry

  lax.fori_loop(0, CH // NBUF, outer, 0)
  plsc.subcore_barrier()
  pltpu.sync_copy(acc.at[pl.ds(r0, RPT)], s_out.at[pl.ds(r0, RPT)])


def _make_agg(nslabs):
  npairs = nslabs // 2

  @functools.partial(
      pl.kernel,
      mesh=_mesh(),
      out_type=tuple(
          jax.ShapeDtypeStruct((NPAD, F), _f32) for _ in range(nslabs)),
      scratch_types=[
          pltpu.VMEM_SHARED((NPAD, F), _f32),
          pltpu.VMEM((CH, K), jnp.int32),
          pltpu.VMEM((NBUF, K), jnp.int32),
          pltpu.VMEM((NBUF, K, F), _f32),
      ] + [pltpu.SemaphoreType.DMA] * (2 * NBUF),
  )
  def agg(*refs):
    us = refs[0:nslabs]
    z = refs[nslabs]
    srcs = refs[nslabs + 1]
    dsts = refs[nslabs + 2]
    outs = refs[nslabs + 3:2 * nslabs + 3]
    acc, src_v, dstr, rows = refs[2 * nslabs + 3:2 * nslabs + 7]
    gsems = refs[2 * nslabs + 7:2 * nslabs + 7 + NBUF]
    dsems = refs[2 * nslabs + 7 + NBUF:]
    c = lax.axis_index("c")
    s = lax.axis_index("s")
    pltpu.sync_copy(srcs.at[s], src_v)
    for p in range(npairs):

      @pl.when(c == 0)
      def _core0():
        _agg_run(us[p], outs[p], z, dsts, src_v, dstr, acc, rows, gsems,
                 dsems, s)

      @pl.when(c == 1)
      def _core1():
        _agg_run(us[npairs + p], outs[npairs + p], z, dsts, src_v, dstr,
                 acc, rows, gsems, dsems, s)

  return agg


@functools.lru_cache(maxsize=None)
def _agg2():
  return _make_agg(2)


@functools.lru_cache(maxsize=None)
def _make_deg():

  @functools.partial(
      pl.kernel,
      mesh=_mesh(),
      out_type=jax.ShapeDtypeStruct((NPAD, F), _f32),
      scratch_types=[
          pltpu.VMEM_SHARED((NPAD, F), _f32),
          pltpu.VMEM((NBUF, K), jnp.int32),
          pltpu.VMEM((K, F), _f32),
      ] + [pltpu.SemaphoreType.DMA] * NBUF,
  )
  def _deg_kernel(ones_hbm, zeros, dsts, degs, acc, dstr, ones_v, *dsems):
    c = lax.axis_index("c")
    s = lax.axis_index("s")

    @pl.when(c == 0)
    def _():
      pltpu.sync_copy(ones_hbm, ones_v)
      r0 = s * RPT
      pltpu.sync_copy(zeros.at[pl.ds(r0, RPT)], acc.at[pl.ds(r0, RPT)])
      plsc.subcore_barrier()
      for b in range(NBUF):
        pltpu.async_copy(dsts.at[s].at[b], dstr.at[b], dsems[b])

      def outer(t, carry):
        for b in range(NBUF):
          j = t * NBUF + b
          pltpu.make_async_copy(dsts.at[s].at[j], dstr.at[b],
                                dsems[b]).wait()
          pltpu.sync_copy(ones_v, acc.at[dstr.at[b]], add=True)
          jn = j + NBUF

          @pl.when(jn < CH)
          def _fire():
            pltpu.async_copy(dsts.at[s].at[jn], dstr.at[b], dsems[b])
        return carry

      lax.fori_loop(0, CH // NBUF, outer, 0)
      plsc.subcore_barrier()
      pltpu.sync_copy(acc.at[pl.ds(r0, RPT)], degs.at[pl.ds(r0, RPT)])

  return _deg_kernel


def _row_spec(w):
  return pl.BlockSpec((R, w), lambda i: (i, 0))


def _full_spec(shape):
  return pl.BlockSpec(shape, lambda i: tuple(0 for _ in shape))


def _prep_body(deg_ref, x_ref, u1a_ref, u1b_ref, dinv_ref):
  d = lax.rsqrt(deg_ref[:, 0:1] + 1.0)
  u = x_ref[...] * d
  u1a_ref[...] = u[:, :F]
  u1b_ref[...] = u[:, F:]
  dinv_ref[...] = jnp.broadcast_to(d, (R, F))


def _prep(degs, xp):
  return pl.pallas_call(
      _prep_body,
      grid=(GRID,),
      in_specs=[_row_spec(F), _row_spec(256)],
      out_specs=[_row_spec(F), _row_spec(F), _row_spec(F)],
      out_shape=[jax.ShapeDtypeStruct((NPAD, F), _f32)] * 3,
  )(degs, xp)


def _d1_body(s1a, s1b, u1a, u1b, dinv, w1, b1, o0, o1, o2, o3):
  dv = dinv[:, 0:1]
  a = jnp.concatenate(
      [(s1a[...] + u1a[...]) * dv, (s1b[...] + u1b[...]) * dv], axis=1)
  h = jnp.dot(a, w1[...], preferred_element_type=_f32) + b1[...]
  u2 = jnp.maximum(h, 0.0) * dv
  o0[...] = u2[:, 0 * F:1 * F]
  o1[...] = u2[:, 1 * F:2 * F]
  o2[...] = u2[:, 2 * F:3 * F]
  o3[...] = u2[:, 3 * F:4 * F]


def _d1(s1a, s1b, u1a, u1b, dinv, W1, b1):
  return pl.pallas_call(
      _d1_body,
      grid=(GRID,),
      in_specs=[_row_spec(F)] * 5 + [_full_spec((256, 512)),
                                     _full_spec((1, 512))],
      out_specs=[_row_spec(F)] * 4,
      out_shape=[jax.ShapeDtypeStruct((NPAD, F), _f32)] * 4,
  )(s1a, s1b, u1a, u1b, dinv, W1, b1)


def _d2_body(s2a, s2b, s2c, s2d, u2a, u2b, u2c, u2d, dinv, w2, b2, w3,
             o0, o1):
  dv = dinv[:, 0:1]
  a = jnp.concatenate(
      [(s2a[...] + u2a[...]), (s2b[...] + u2b[...]),
       (s2c[...] + u2c[...]), (s2d[...] + u2d[...])], axis=1) * dv
  h = jnp.dot(a, w2[...], preferred_element_type=_f32) + b2[...]
  h = jnp.maximum(h, 0.0)
  u3 = jnp.dot(h, w3[...], preferred_element_type=_f32) * dv
  o0[...] = u3[:, :F]
  o1[...] = u3[:, F:]


def _d2(s2, u2, dinv, W2, b2, W3):
  return pl.pallas_call(
      _d2_body,
      grid=(GRID,),
      in_specs=[_row_spec(F)] * 9 + [_full_spec((512, 512)),
                                     _full_spec((1, 512)),
                                     _full_spec((512, 256))],
      out_specs=[_row_spec(F)] * 2,
      out_shape=[jax.ShapeDtypeStruct((NPAD, F), _f32)] * 2,
  )(*s2, *u2, dinv, W2, b2, W3)


def _d3_body(s3a, s3b, u3a, u3b, dinv, b3, out_ref):
  dv = dinv[:, 0:1]
  a = jnp.concatenate(
      [(s3a[...] + u3a[...]), (s3b[...] + u3b[...])], axis=1) * dv
  out_ref[...] = a + b3[...]


def _d3(s3a, s3b, u3a, u3b, dinv, b3):
  return pl.pallas_call(
      _d3_body,
      grid=(GRID,),
      in_specs=[_row_spec(F)] * 5 + [_full_spec((1, 256))],
      out_specs=_row_spec(256),
      out_shape=jax.ShapeDtypeStruct((NPAD, 256), _f32),
  )(s3a, s3b, u3a, u3b, dinv, b3)


@jax.jit
def _run(x, edge_index, W1, b1, W2, b2, W3, b3):
  src = edge_index[0].astype(jnp.int32)
  dst = edge_index[1].astype(jnp.int32)
  pad = EPAD - E
  srcp = jnp.concatenate([src, jnp.zeros((pad,), jnp.int32)])
  dstp = jnp.concatenate([dst, jnp.full((pad,), N, jnp.int32)])
  srcp = srcp.reshape(TILES, CH, K)
  dstp = dstp.reshape(TILES, CH, K)
  xp = jnp.pad(x, ((0, NPAD - N), (0, 0)))
  z = jnp.zeros((NPAD, F), _f32)
  ones = jnp.ones((K, F), _f32)

  agg2 = _agg2()
  degs = _make_deg()(ones, z, dstp)
  u1a, u1b, dinv = _prep(degs, xp)
  s1a, s1b = agg2(u1a, u1b, z, srcp, dstp)
  u2 = _d1(s1a, s1b, u1a, u1b, dinv, W1, b1.reshape(1, 512))
  s2ab = agg2(u2[0], u2[1], z, srcp, dstp)
  s2cd = agg2(u2[2], u2[3], z, srcp, dstp)
  u3a, u3b = _d2((*s2ab, *s2cd), u2, dinv, W2, b2.reshape(1, 512), W3)
  s3a, s3b = agg2(u3a, u3b, z, srcp, dstp)
  out = _d3(s3a, s3b, u3a, u3b, dinv, b3.reshape(1, 256))
  return out[:N]


def kernel(x, edge_index, W1, b1, W2, b2, W3, b3):
  return _run(x, edge_index, W1, b1, W2, b2, W3, b3)
